# baseline (speedup 1.0000x reference)
import jax
import jax.numpy as jnp
from jax import lax
from jax.experimental import pallas as pl
from jax.experimental.pallas import tpu as pltpu
from jax.experimental.pallas import tpu_sc as plsc

N = 10000
E = 320000
D = 128
LAT = 128
H = 2 * LAT

NC = 2
NS = 16
NW = NC * NS
EW = E // NW
K = 80
NCHUNK = EW // K
RPT = 632
RPT_LAST = N - 15 * RPT

_mesh = plsc.VectorSubcoreMesh(core_axis_name="c", subcore_axis_name="s")


def _edge_scatter_body(ea0_hbm, ea1_hbm, dst_hbm, out0_hbm, out1_hbm,
                       acc0, acc1, dst_v, ea0_v, ea1_v):
    c = lax.axis_index("c")
    s = lax.axis_index("s")
    w = c * NS + s
    e0 = pl.multiple_of(w * EW, 8)
    pltpu.sync_copy(dst_hbm.at[pl.ds(e0, EW)], dst_v)
    pltpu.sync_copy(ea0_hbm.at[pl.ds(e0, EW)], ea0_v)
    pltpu.sync_copy(ea1_hbm.at[pl.ds(e0, EW)], ea1_v)

    z16 = jnp.zeros((16,), jnp.float32)

    def zbody(k, carry):
        acc0[pl.ds(k * 16, 16)] = z16
        acc1[pl.ds(k * 16, 16)] = z16
        return carry

    lax.fori_loop(0, N // 16, zbody, 0)

    def body(k, carry):
        o = k * 16
        idx = dst_v[pl.ds(o, 16)]
        plsc.addupdate_scatter(acc0, [idx], ea0_v[pl.ds(o, 16)])
        plsc.addupdate_scatter(acc1, [idx], ea1_v[pl.ds(o, 16)])
        return carry

    lax.fori_loop(0, EW // 16, body, 0)
    o0 = pl.multiple_of(w * N, 8)
    pltpu.sync_copy(acc0, out0_hbm.at[pl.ds(o0, N)])
    pltpu.sync_copy(acc1, out1_hbm.at[pl.ds(o0, N)])


def _edge_scatter(ea0, ea1, dst):
    return pl.kernel(
        _edge_scatter_body,
        out_type=[
            jax.ShapeDtypeStruct((NW * N,), jnp.float32),
            jax.ShapeDtypeStruct((NW * N,), jnp.float32),
        ],
        mesh=_mesh,
        compiler_params=pltpu.CompilerParams(needs_layout_passes=False),
        scratch_types=[
            pltpu.VMEM((N,), jnp.float32),
            pltpu.VMEM((N,), jnp.float32),
            pltpu.VMEM((EW,), jnp.int32),
            pltpu.VMEM((EW,), jnp.float32),
            pltpu.VMEM((EW,), jnp.float32),
        ],
    )(ea0, ea1, dst)


KS = 128
NR = 4
RC = 40
ETP = NR * RC * KS
N_ACC = N + 8


def _segsum_body(tbl_hbm, srcs_hbm, dsti_hbm, z_hbm, out_hbm,
                 acc, idx_s, idx_d, rows0, rows1, sem):
    c = lax.axis_index("c")
    s = lax.axis_index("s")
    row0 = pl.multiple_of(s * RPT, 8)
    tile_base = pl.multiple_of((c * NS + s) * ETP, 8)

    @pl.when(s < 15)
    def _():
        pltpu.sync_copy(z_hbm.at[pl.ds(0, RPT)], acc.at[pl.ds(row0, RPT)])

    @pl.when(s == 15)
    def _():
        pltpu.sync_copy(z_hbm.at[pl.ds(0, RPT_LAST)],
                        acc.at[pl.ds(15 * RPT, RPT_LAST)])

    plsc.subcore_barrier()

    def gather(j, buf):
        jo = pl.multiple_of(j * KS, 8)
        return pltpu.async_copy(tbl_hbm.at[idx_s.at[pl.ds(jo, KS)]], buf, sem)

    def gather_wait(j, buf):
        jo = pl.multiple_of(j * KS, 8)
        pltpu.make_async_copy(tbl_hbm.at[idx_s.at[pl.ds(jo, KS)]], buf,
                              sem).wait()

    for r in range(NR):
        pltpu.sync_copy(
            srcs_hbm.at[pl.ds(tile_base + r * RC * KS, RC * KS)], idx_s)
        pltpu.sync_copy(dsti_hbm.at[s, r], idx_d)
        gather(0, rows0)

        def body(jj, carry):
            j0 = 2 * jj
            gather_wait(j0, rows0)
            gather(j0 + 1, rows1)
            pltpu.sync_copy(rows0, acc.at[idx_d.at[j0]], add=True)
            gather_wait(j0 + 1, rows1)

            @pl.when(jj < RC // 2 - 1)
            def _():
                gather(j0 + 2, rows0)

            pltpu.sync_copy(rows1, acc.at[idx_d.at[j0 + 1]], add=True)
            return carry

        lax.fori_loop(0, RC // 2, body, 0)

    plsc.subcore_barrier()

    @pl.when(s < 15)
    def _():
        pltpu.sync_copy(acc.at[pl.ds(row0, RPT)],
                        out_hbm.at[c, pl.ds(row0, RPT)])

    @pl.when(s == 15)
    def _():
        pltpu.sync_copy(acc.at[pl.ds(15 * RPT, RPT_LAST)],
                        out_hbm.at[c, pl.ds(15 * RPT, RPT_LAST)])


def _segsum(tbl_flat, srcs, dsti, zrows):
    return pl.kernel(
        _segsum_body,
        out_type=jax.ShapeDtypeStruct((NC, N, D), jnp.float32),
        mesh=_mesh,
        compiler_params=pltpu.CompilerParams(needs_layout_passes=False),
        scratch_types=[
            pltpu.VMEM_SHARED((N_ACC, D), jnp.float32),
            pltpu.VMEM((RC * KS,), jnp.int32),
            pltpu.VMEM((RC, KS), jnp.int32),
            pltpu.VMEM((KS, D), jnp.float32),
            pltpu.VMEM((KS, D), jnp.float32),
            pltpu.SemaphoreType.DMA,
        ],
    )(tbl_flat, srcs, dsti, zrows)


_BN = 1000


def _input_pot_body(x_ref, sa_ref, sb_ref, wn_ref, we0_ref, we1_ref, b_ref,
                    ip_ref, tbl_ref):
    sa = jnp.sum(sa_ref[...], axis=1)[:, None]
    sb = jnp.sum(sb_ref[...], axis=1)[:, None]
    t = jnp.dot(x_ref[...], wn_ref[...], preferred_element_type=jnp.float32)
    t += sa * we0_ref[...] + sb * we1_ref[...]
    ip = jnp.maximum(t + b_ref[...], 0.0)
    ip_ref[...] = ip
    tbl_ref[0] = ip[:, :D]
    tbl_ref[1] = ip[:, D:]


def _input_potential(x, sa, sb, w_n2l, we0, we1, b_n2l):
    return pl.pallas_call(
        _input_pot_body,
        grid=(N // _BN,),
        in_specs=[
            pl.BlockSpec((_BN, D), lambda i: (i, 0)),
            pl.BlockSpec((_BN, NW), lambda i: (i, 0)),
            pl.BlockSpec((_BN, NW), lambda i: (i, 0)),
            pl.BlockSpec((D, H), lambda i: (0, 0)),
            pl.BlockSpec((1, H), lambda i: (0, 0)),
            pl.BlockSpec((1, H), lambda i: (0, 0)),
            pl.BlockSpec((1, H), lambda i: (0, 0)),
        ],
        out_specs=[
            pl.BlockSpec((_BN, H), lambda i: (i, 0)),
            pl.BlockSpec((NC, _BN, D), lambda i: (0, i, 0)),
        ],
        out_shape=[
            jax.ShapeDtypeStruct((N, H), jnp.float32),
            jax.ShapeDtypeStruct((NC, N, D), jnp.float32),
        ],
    )(x, sa, sb, w_n2l, we0, we1, b_n2l)


def _layer_body(p_ref, ip_ref, w0_ref, w1_ref, b_ref,
                tbl_ref, g_ref):
    i = pl.program_id(0)
    a0 = p_ref[0]
    a1 = p_ref[1]
    t = jnp.dot(a0, w0_ref[...], preferred_element_type=jnp.float32)
    t += jnp.dot(a1, w1_ref[...], preferred_element_type=jnp.float32)
    cur = jnp.maximum(t + b_ref[...] + ip_ref[...], 0.0)
    tbl_ref[0] = cur[:, :D]
    tbl_ref[1] = cur[:, D:]

    @pl.when(i == 0)
    def _():
        g_ref[...] = jnp.zeros_like(g_ref)

    g_ref[...] += jnp.sum(cur, axis=0, keepdims=True)


def _layer_update(parts, ip, conv_W0, conv_W1, conv_b):
    return pl.pallas_call(
        _layer_body,
        grid=(N // _BN,),
        in_specs=[
            pl.BlockSpec((NC, _BN, D), lambda i: (0, i, 0)),
            pl.BlockSpec((_BN, H), lambda i: (i, 0)),
            pl.BlockSpec((D, H), lambda i: (0, 0)),
            pl.BlockSpec((D, H), lambda i: (0, 0)),
            pl.BlockSpec((1, H), lambda i: (0, 0)),
        ],
        out_specs=[
            pl.BlockSpec((NC, _BN, D), lambda i: (0, i, 0)),
            pl.BlockSpec((1, H), lambda i: (0, 0)),
        ],
        out_shape=[
            jax.ShapeDtypeStruct((NC, N, D), jnp.float32),
            jax.ShapeDtypeStruct((1, H), jnp.float32),
        ],
    )(parts, ip, conv_W0, conv_W1, conv_b)


def _head_body(graph_ref, bud_ref, gw_ref, bw_ref, cb_ref,
               s1w_ref, s1bw_ref, s1b_ref, soutw_ref, soutb_ref,
               nw_ref, nb_ref, a1g_ref, a1ng_ref, a1b_ref,
               sv_ref, c0_ref):
    bud = bud_ref[...]
    g = jnp.dot(graph_ref[...], gw_ref[...], preferred_element_type=jnp.float32)
    g = jnp.maximum(g + bud * bw_ref[...] + cb_ref[...], 0.0)
    sh = jnp.dot(g, s1w_ref[...], preferred_element_type=jnp.float32)
    sh = jnp.maximum(sh + bud * s1bw_ref[...] + s1b_ref[...], 0.0)
    sv_ref[...] = (jnp.dot(sh, soutw_ref[...], preferred_element_type=jnp.float32)
                   + soutb_ref[...])
    ng = jnp.dot(g, nw_ref[...], preferred_element_type=jnp.float32)
    ng = jnp.maximum(ng + nb_ref[...], 0.0)
    c0 = jnp.dot(g, a1g_ref[...], preferred_element_type=jnp.float32)
    c0 += jnp.dot(ng, a1ng_ref[...], preferred_element_type=jnp.float32)
    c0_ref[...] = c0 + a1b_ref[...]


def _head(graph, budget, gW, bW, cb, s1W, s1bw, s1b, soutW, soutb,
          nodesW, nodesb, A1g, A1ng, a1b):
    return pl.pallas_call(
        _head_body,
        out_shape=[
            jax.ShapeDtypeStruct((1, 1), jnp.float32),
            jax.ShapeDtypeStruct((1, 64), jnp.float32),
        ],
    )(graph, budget, gW, bW, cb, s1W, s1bw, s1b, soutW, soutb,
      nodesW, nodesb, A1g, A1ng, a1b)


_BE = 4000


def _q_body(wt_ref, c0_ref, ww_ref, wb_ref, a1m_ref, a384_ref,
            aoutw_ref, aoutb_ref, q_ref):
    wt = wt_ref[...]
    wtemb = jnp.maximum(wt * ww_ref[...] + wb_ref[...], 0.0)
    mid = jnp.dot(wtemb, a1m_ref[...], preferred_element_type=jnp.float32)
    ah = jnp.maximum(mid + c0_ref[...] + wt * a384_ref[...], 0.0)
    q_ref[...] = (jnp.dot(ah, aoutw_ref[...], preferred_element_type=jnp.float32)
                  + aoutb_ref[...])


def _q_head(wt, c0, wW, wb, A1mid, a384, aoutW, aoutb):
    return pl.pallas_call(
        _q_body,
        grid=(E // _BE,),
        in_specs=[
            pl.BlockSpec((_BE, 1), lambda i: (i, 0)),
            pl.BlockSpec((1, 64), lambda i: (0, 0)),
            pl.BlockSpec((1, LAT), lambda i: (0, 0)),
            pl.BlockSpec((1, LAT), lambda i: (0, 0)),
            pl.BlockSpec((LAT, 64), lambda i: (0, 0)),
            pl.BlockSpec((1, 64), lambda i: (0, 0)),
            pl.BlockSpec((64, 1), lambda i: (0, 0)),
            pl.BlockSpec((1, 1), lambda i: (0, 0)),
        ],
        out_specs=pl.BlockSpec((_BE, 1), lambda i: (i, 0)),
        out_shape=jax.ShapeDtypeStruct((E, 1), jnp.float32),
    )(wt, c0, wW, wb, A1mid, a384, aoutW, aoutb)


def kernel(x, edge_index, edge_attr, budget, w_n2l, b_n2l, w_e2l, conv_W,
           conv_b, concat_W, concat_b, weight_W, weight_b, nodes_W, nodes_b,
           s1_W, s1_b, sout_W, sout_b, a1_W, a1_b, aout_W, aout_b):
    f32 = jnp.float32
    src = edge_index[0].astype(jnp.int32)
    dst = edge_index[1].astype(jnp.int32)
    src2 = jnp.concatenate(
        [src.reshape(NS, E // NS),
         jnp.zeros((NS, ETP - E // NS), jnp.int32)], axis=1)
    srcs = jnp.concatenate(
        [src2.reshape(-1), src2.reshape(-1) + N])
    dst2 = jnp.concatenate(
        [dst.reshape(NS, E // NS),
         jnp.full((NS, ETP - E // NS), N, jnp.int32)], axis=1)
    dsti = dst2.reshape(NS, NR, RC, KS)

    ea0 = edge_attr[:, 0]
    ea1 = edge_attr[:, 1]
    z128 = jnp.zeros((RPT, D), f32)

    s0f, s1f = _edge_scatter(ea0, ea1, dst)
    sa = s0f.reshape(NW, N).T
    sb = s1f.reshape(NW, N).T
    ip, tbl = _input_potential(
        x, sa, sb, w_n2l, w_e2l[0].reshape(1, H), w_e2l[1].reshape(1, H),
        b_n2l.reshape(1, H))

    conv_W0 = conv_W[:D]
    conv_W1 = conv_W[D:]
    cbr = conv_b.reshape(1, H)
    graph = None
    for _ in range(3):
        pool = _segsum(tbl.reshape(NC * N, D), srcs, dsti, z128)
        tbl, graph = _layer_update(pool, ip, conv_W0, conv_W1, cbr)

    state_val, c0 = _head(
        graph, budget.reshape(1, 1),
        concat_W[:H], concat_W[H:], concat_b.reshape(1, LAT),
        s1_W[:LAT], s1_W[LAT:], s1_b.reshape(1, 64),
        sout_W, sout_b.reshape(1, 1),
        nodes_W, nodes_b.reshape(1, LAT),
        a1_W[:LAT], a1_W[2 * LAT:3 * LAT], a1_b.reshape(1, 64))

    q = _q_head(edge_attr[:, :1], c0, weight_W, weight_b.reshape(1, LAT),
                a1_W[LAT:2 * LAT], a1_W[3 * LAT:], aout_W, aout_b.reshape(1, 1))

    return jnp.concatenate([state_val, q], axis=0)


# speedup vs baseline: 1.5408x; 1.5408x over previous
import jax
import jax.numpy as jnp
from jax import lax
from jax.experimental import pallas as pl
from jax.experimental.pallas import tpu as pltpu
from jax.experimental.pallas import tpu_sc as plsc

N = 10000
E = 320000
D = 128
LAT = 128
H = 2 * LAT

NC = 2
NS = 16
NW = NC * NS
EW = E // NW
K = 80
NCHUNK = EW // K
RPT = 632
RPT_LAST = N - 15 * RPT

_mesh = plsc.VectorSubcoreMesh(core_axis_name="c", subcore_axis_name="s")


def _edge_scatter_body(ea0_hbm, ea1_hbm, dst_hbm, out0_hbm, out1_hbm,
                       acc0, acc1, dst_v, ea0_v, ea1_v):
    c = lax.axis_index("c")
    s = lax.axis_index("s")
    w = c * NS + s
    e0 = pl.multiple_of(w * EW, 8)
    pltpu.sync_copy(dst_hbm.at[pl.ds(e0, EW)], dst_v)
    pltpu.sync_copy(ea0_hbm.at[pl.ds(e0, EW)], ea0_v)
    pltpu.sync_copy(ea1_hbm.at[pl.ds(e0, EW)], ea1_v)

    z16 = jnp.zeros((16,), jnp.float32)

    def zbody(k, carry):
        acc0[pl.ds(k * 16, 16)] = z16
        acc1[pl.ds(k * 16, 16)] = z16
        return carry

    lax.fori_loop(0, N // 16, zbody, 0)

    def body(k, carry):
        o = k * 16
        idx = dst_v[pl.ds(o, 16)]
        plsc.addupdate_scatter(acc0, [idx], ea0_v[pl.ds(o, 16)])
        plsc.addupdate_scatter(acc1, [idx], ea1_v[pl.ds(o, 16)])
        return carry

    lax.fori_loop(0, EW // 16, body, 0)
    o0 = pl.multiple_of(w * N, 8)
    pltpu.sync_copy(acc0, out0_hbm.at[pl.ds(o0, N)])
    pltpu.sync_copy(acc1, out1_hbm.at[pl.ds(o0, N)])


def _edge_scatter(ea0, ea1, dst):
    return pl.kernel(
        _edge_scatter_body,
        out_type=[
            jax.ShapeDtypeStruct((NW * N,), jnp.float32),
            jax.ShapeDtypeStruct((NW * N,), jnp.float32),
        ],
        mesh=_mesh,
        compiler_params=pltpu.CompilerParams(needs_layout_passes=False),
        scratch_types=[
            pltpu.VMEM((N,), jnp.float32),
            pltpu.VMEM((N,), jnp.float32),
            pltpu.VMEM((EW,), jnp.int32),
            pltpu.VMEM((EW,), jnp.float32),
            pltpu.VMEM((EW,), jnp.float32),
        ],
    )(ea0, ea1, dst)


KS = 80
NR = 5
RC = 50
ETP = NR * RC * KS
N_ACC = N


def _segsum_body(tbl_hbm, srcs_hbm, dsti_hbm, z_hbm, out_hbm,
                 acc, idx_s, idx_d, rows0, rows1, sem):
    c = lax.axis_index("c")
    s = lax.axis_index("s")
    row0 = pl.multiple_of(s * RPT, 8)
    tile_base = pl.multiple_of((c * NS + s) * ETP, 8)
    pltpu.sync_copy(srcs_hbm.at[pl.ds(tile_base, ETP)], idx_s)

    @pl.when(s < 15)
    def _():
        pltpu.sync_copy(z_hbm.at[pl.ds(0, RPT)], acc.at[pl.ds(row0, RPT)])

    @pl.when(s == 15)
    def _():
        pltpu.sync_copy(z_hbm.at[pl.ds(0, RPT_LAST)],
                        acc.at[pl.ds(15 * RPT, RPT_LAST)])

    plsc.subcore_barrier()

    def gather(j, buf):
        jo = pl.multiple_of(j * KS, 8)
        return pltpu.async_copy(tbl_hbm.at[idx_s.at[pl.ds(jo, KS)]], buf, sem)

    def gather_wait(j, buf):
        jo = pl.multiple_of(j * KS, 8)
        pltpu.make_async_copy(tbl_hbm.at[idx_s.at[pl.ds(jo, KS)]], buf,
                              sem).wait()

    for r in range(NR):
        pltpu.sync_copy(dsti_hbm.at[s, r], idx_d)
        gather(r * RC, rows0)

        def body(jj, carry):
            j0 = 2 * jj
            g0 = r * RC + j0
            gather_wait(g0, rows0)
            gather(g0 + 1, rows1)
            pltpu.sync_copy(rows0, acc.at[idx_d.at[j0]], add=True)
            gather_wait(g0 + 1, rows1)

            @pl.when(jj < RC // 2 - 1)
            def _():
                gather(g0 + 2, rows0)

            pltpu.sync_copy(rows1, acc.at[idx_d.at[j0 + 1]], add=True)
            return carry

        lax.fori_loop(0, RC // 2, body, 0)

    plsc.subcore_barrier()

    @pl.when(s < 15)
    def _():
        pltpu.sync_copy(acc.at[pl.ds(row0, RPT)],
                        out_hbm.at[c, pl.ds(row0, RPT)])

    @pl.when(s == 15)
    def _():
        pltpu.sync_copy(acc.at[pl.ds(15 * RPT, RPT_LAST)],
                        out_hbm.at[c, pl.ds(15 * RPT, RPT_LAST)])


def _segsum(tbl_flat, srcs, dsti, zrows):
    return pl.kernel(
        _segsum_body,
        out_type=jax.ShapeDtypeStruct((NC, N, D), jnp.float32),
        mesh=_mesh,
        compiler_params=pltpu.CompilerParams(needs_layout_passes=False),
        scratch_types=[
            pltpu.VMEM_SHARED((N_ACC, D), jnp.float32),
            pltpu.VMEM((ETP,), jnp.int32),
            pltpu.VMEM((RC, KS), jnp.int32),
            pltpu.VMEM((KS, D), jnp.float32),
            pltpu.VMEM((KS, D), jnp.float32),
            pltpu.SemaphoreType.DMA,
        ],
    )(tbl_flat, srcs, dsti, zrows)


_BN = 1000


def _input_pot_body(x_ref, sa_ref, sb_ref, wn_ref, we0_ref, we1_ref, b_ref,
                    ip_ref, tbl_ref):
    sa = jnp.sum(sa_ref[...], axis=1)[:, None]
    sb = jnp.sum(sb_ref[...], axis=1)[:, None]
    t = jnp.dot(x_ref[...], wn_ref[...], preferred_element_type=jnp.float32)
    t += sa * we0_ref[...] + sb * we1_ref[...]
    ip = jnp.maximum(t + b_ref[...], 0.0)
    ip_ref[...] = ip
    tbl_ref[0] = ip[:, :D]
    tbl_ref[1] = ip[:, D:]


def _input_potential(x, sa, sb, w_n2l, we0, we1, b_n2l):
    return pl.pallas_call(
        _input_pot_body,
        grid=(N // _BN,),
        in_specs=[
            pl.BlockSpec((_BN, D), lambda i: (i, 0)),
            pl.BlockSpec((_BN, NW), lambda i: (i, 0)),
            pl.BlockSpec((_BN, NW), lambda i: (i, 0)),
            pl.BlockSpec((D, H), lambda i: (0, 0)),
            pl.BlockSpec((1, H), lambda i: (0, 0)),
            pl.BlockSpec((1, H), lambda i: (0, 0)),
            pl.BlockSpec((1, H), lambda i: (0, 0)),
        ],
        out_specs=[
            pl.BlockSpec((_BN, H), lambda i: (i, 0)),
            pl.BlockSpec((NC, _BN, D), lambda i: (0, i, 0)),
        ],
        out_shape=[
            jax.ShapeDtypeStruct((N, H), jnp.float32),
            jax.ShapeDtypeStruct((NC, N, D), jnp.float32),
        ],
    )(x, sa, sb, w_n2l, we0, we1, b_n2l)


def _layer_body(p_ref, ip_ref, w0_ref, w1_ref, b_ref,
                tbl_ref, g_ref):
    i = pl.program_id(0)
    a0 = p_ref[0]
    a1 = p_ref[1]
    t = jnp.dot(a0, w0_ref[...], preferred_element_type=jnp.float32)
    t += jnp.dot(a1, w1_ref[...], preferred_element_type=jnp.float32)
    cur = jnp.maximum(t + b_ref[...] + ip_ref[...], 0.0)
    tbl_ref[0] = cur[:, :D]
    tbl_ref[1] = cur[:, D:]

    @pl.when(i == 0)
    def _():
        g_ref[...] = jnp.zeros_like(g_ref)

    g_ref[...] += jnp.sum(cur, axis=0, keepdims=True)


def _layer_update(parts, ip, conv_W0, conv_W1, conv_b):
    return pl.pallas_call(
        _layer_body,
        grid=(N // _BN,),
        in_specs=[
            pl.BlockSpec((NC, _BN, D), lambda i: (0, i, 0)),
            pl.BlockSpec((_BN, H), lambda i: (i, 0)),
            pl.BlockSpec((D, H), lambda i: (0, 0)),
            pl.BlockSpec((D, H), lambda i: (0, 0)),
            pl.BlockSpec((1, H), lambda i: (0, 0)),
        ],
        out_specs=[
            pl.BlockSpec((NC, _BN, D), lambda i: (0, i, 0)),
            pl.BlockSpec((1, H), lambda i: (0, 0)),
        ],
        out_shape=[
            jax.ShapeDtypeStruct((NC, N, D), jnp.float32),
            jax.ShapeDtypeStruct((1, H), jnp.float32),
        ],
    )(parts, ip, conv_W0, conv_W1, conv_b)


def _head_body(graph_ref, bud_ref, gw_ref, bw_ref, cb_ref,
               s1w_ref, s1bw_ref, s1b_ref, soutw_ref, soutb_ref,
               nw_ref, nb_ref, a1g_ref, a1ng_ref, a1b_ref,
               sv_ref, c0_ref):
    bud = bud_ref[...]
    g = jnp.dot(graph_ref[...], gw_ref[...], preferred_element_type=jnp.float32)
    g = jnp.maximum(g + bud * bw_ref[...] + cb_ref[...], 0.0)
    sh = jnp.dot(g, s1w_ref[...], preferred_element_type=jnp.float32)
    sh = jnp.maximum(sh + bud * s1bw_ref[...] + s1b_ref[...], 0.0)
    sv_ref[...] = (jnp.dot(sh, soutw_ref[...], preferred_element_type=jnp.float32)
                   + soutb_ref[...])
    ng = jnp.dot(g, nw_ref[...], preferred_element_type=jnp.float32)
    ng = jnp.maximum(ng + nb_ref[...], 0.0)
    c0 = jnp.dot(g, a1g_ref[...], preferred_element_type=jnp.float32)
    c0 += jnp.dot(ng, a1ng_ref[...], preferred_element_type=jnp.float32)
    c0_ref[...] = c0 + a1b_ref[...]


def _head(graph, budget, gW, bW, cb, s1W, s1bw, s1b, soutW, soutb,
          nodesW, nodesb, A1g, A1ng, a1b):
    return pl.pallas_call(
        _head_body,
        out_shape=[
            jax.ShapeDtypeStruct((1, 1), jnp.float32),
            jax.ShapeDtypeStruct((1, 64), jnp.float32),
        ],
    )(graph, budget, gW, bW, cb, s1W, s1bw, s1b, soutW, soutb,
      nodesW, nodesb, A1g, A1ng, a1b)


_BE = 4000


def _q_body(wt_ref, c0_ref, ww_ref, wb_ref, a1m_ref, a384_ref,
            aoutw_ref, aoutb_ref, q_ref):
    wt = wt_ref[...]
    wtemb = jnp.maximum(wt * ww_ref[...] + wb_ref[...], 0.0)
    mid = jnp.dot(wtemb, a1m_ref[...], preferred_element_type=jnp.float32)
    ah = jnp.maximum(mid + c0_ref[...] + wt * a384_ref[...], 0.0)
    q_ref[...] = (jnp.dot(ah, aoutw_ref[...], preferred_element_type=jnp.float32)
                  + aoutb_ref[...])


def _q_head(wt, c0, wW, wb, A1mid, a384, aoutW, aoutb):
    return pl.pallas_call(
        _q_body,
        grid=(E // _BE,),
        in_specs=[
            pl.BlockSpec((_BE, 1), lambda i: (i, 0)),
            pl.BlockSpec((1, 64), lambda i: (0, 0)),
            pl.BlockSpec((1, LAT), lambda i: (0, 0)),
            pl.BlockSpec((1, LAT), lambda i: (0, 0)),
            pl.BlockSpec((LAT, 64), lambda i: (0, 0)),
            pl.BlockSpec((1, 64), lambda i: (0, 0)),
            pl.BlockSpec((64, 1), lambda i: (0, 0)),
            pl.BlockSpec((1, 1), lambda i: (0, 0)),
        ],
        out_specs=pl.BlockSpec((_BE, 1), lambda i: (i, 0)),
        out_shape=jax.ShapeDtypeStruct((E, 1), jnp.float32),
    )(wt, c0, wW, wb, A1mid, a384, aoutW, aoutb)


def kernel(x, edge_index, edge_attr, budget, w_n2l, b_n2l, w_e2l, conv_W,
           conv_b, concat_W, concat_b, weight_W, weight_b, nodes_W, nodes_b,
           s1_W, s1_b, sout_W, sout_b, a1_W, a1_b, aout_W, aout_b):
    f32 = jnp.float32
    src = edge_index[0].astype(jnp.int32)
    dst = edge_index[1].astype(jnp.int32)
    srcs = jnp.concatenate([src, src + N])
    dsti = dst.reshape(NS, NR, RC, KS)

    ea0 = edge_attr[:, 0]
    ea1 = edge_attr[:, 1]
    z128 = jnp.zeros((RPT, D), f32)

    s0f, s1f = _edge_scatter(ea0, ea1, dst)
    sa = s0f.reshape(NW, N).T
    sb = s1f.reshape(NW, N).T
    ip, tbl = _input_potential(
        x, sa, sb, w_n2l, w_e2l[0].reshape(1, H), w_e2l[1].reshape(1, H),
        b_n2l.reshape(1, H))

    conv_W0 = conv_W[:D]
    conv_W1 = conv_W[D:]
    cbr = conv_b.reshape(1, H)
    graph = None
    for _ in range(3):
        pool = _segsum(tbl.reshape(NC * N, D), srcs, dsti, z128)
        tbl, graph = _layer_update(pool, ip, conv_W0, conv_W1, cbr)

    state_val, c0 = _head(
        graph, budget.reshape(1, 1),
        concat_W[:H], concat_W[H:], concat_b.reshape(1, LAT),
        s1_W[:LAT], s1_W[LAT:], s1_b.reshape(1, 64),
        sout_W, sout_b.reshape(1, 1),
        nodes_W, nodes_b.reshape(1, LAT),
        a1_W[:LAT], a1_W[2 * LAT:3 * LAT], a1_b.reshape(1, 64))

    q = _q_head(edge_attr[:, :1], c0, weight_W, weight_b.reshape(1, LAT),
                a1_W[LAT:2 * LAT], a1_W[3 * LAT:], aout_W, aout_b.reshape(1, 1))

    return jnp.concatenate([state_val, q], axis=0)


# speedup vs baseline: 1.8676x; 1.2121x over previous
import jax
import jax.numpy as jnp
from jax import lax
from jax.experimental import pallas as pl
from jax.experimental.pallas import tpu as pltpu
from jax.experimental.pallas import tpu_sc as plsc

N = 10000
E = 320000
D = 128
LAT = 128
H = 2 * LAT

NC = 2
NS = 16
NW = NC * NS
EW = E // NW
K = 80
NCHUNK = EW // K
RPT = 632
RPT_LAST = N - 15 * RPT

_mesh = plsc.VectorSubcoreMesh(core_axis_name="c", subcore_axis_name="s")


def _edge_scatter_body(ea0_hbm, ea1_hbm, dst_hbm, out0_hbm, out1_hbm,
                       acc0, acc1, dst_v, ea0_v, ea1_v):
    c = lax.axis_index("c")
    s = lax.axis_index("s")
    w = c * NS + s
    e0 = pl.multiple_of(w * EW, 8)
    pltpu.sync_copy(dst_hbm.at[pl.ds(e0, EW)], dst_v)
    pltpu.sync_copy(ea0_hbm.at[pl.ds(e0, EW)], ea0_v)
    pltpu.sync_copy(ea1_hbm.at[pl.ds(e0, EW)], ea1_v)

    z16 = jnp.zeros((16,), jnp.float32)

    def zbody(k, carry):
        acc0[pl.ds(k * 16, 16)] = z16
        acc1[pl.ds(k * 16, 16)] = z16
        return carry

    lax.fori_loop(0, N // 16, zbody, 0)

    def body(k, carry):
        o = k * 16
        idx = dst_v[pl.ds(o, 16)]
        plsc.addupdate_scatter(acc0, [idx], ea0_v[pl.ds(o, 16)])
        plsc.addupdate_scatter(acc1, [idx], ea1_v[pl.ds(o, 16)])
        return carry

    lax.fori_loop(0, EW // 16, body, 0)
    o0 = pl.multiple_of(w * N, 8)
    pltpu.sync_copy(acc0, out0_hbm.at[pl.ds(o0, N)])
    pltpu.sync_copy(acc1, out1_hbm.at[pl.ds(o0, N)])


def _edge_scatter(ea0, ea1, dst):
    return pl.kernel(
        _edge_scatter_body,
        out_type=[
            jax.ShapeDtypeStruct((NW * N,), jnp.float32),
            jax.ShapeDtypeStruct((NW * N,), jnp.float32),
        ],
        mesh=_mesh,
        compiler_params=pltpu.CompilerParams(needs_layout_passes=False),
        scratch_types=[
            pltpu.VMEM((N,), jnp.float32),
            pltpu.VMEM((N,), jnp.float32),
            pltpu.VMEM((EW,), jnp.int32),
            pltpu.VMEM((EW,), jnp.float32),
            pltpu.VMEM((EW,), jnp.float32),
        ],
    )(ea0, ea1, dst)


KS = 80
NR = 5
RC = 50
ETP = NR * RC * KS
N_ACC = N


def _segsum_body(tbl_hbm, srcs_hbm, dsti_hbm, z_hbm, out_hbm,
                 acc, idx_s, idx_d, rows0, rows1, sem, sem2):
    c = lax.axis_index("c")
    s = lax.axis_index("s")
    row0 = pl.multiple_of(s * RPT, 8)
    tile_base = pl.multiple_of((c * NS + s) * ETP, 8)
    pltpu.sync_copy(srcs_hbm.at[pl.ds(tile_base, ETP)], idx_s)

    @pl.when(s < 15)
    def _():
        pltpu.sync_copy(z_hbm.at[pl.ds(0, RPT)], acc.at[pl.ds(row0, RPT)])

    @pl.when(s == 15)
    def _():
        pltpu.sync_copy(z_hbm.at[pl.ds(0, RPT_LAST)],
                        acc.at[pl.ds(15 * RPT, RPT_LAST)])

    plsc.subcore_barrier()

    def gather(j, buf):
        jo = pl.multiple_of(j * KS, 8)
        return pltpu.async_copy(tbl_hbm.at[idx_s.at[pl.ds(jo, KS)]], buf, sem)

    def gather_wait(j, buf):
        jo = pl.multiple_of(j * KS, 8)
        pltpu.make_async_copy(tbl_hbm.at[idx_s.at[pl.ds(jo, KS)]], buf,
                              sem).wait()

    def scatter(j, buf):
        return pltpu.async_copy(buf, acc.at[idx_d.at[j]], sem2, add=True)

    def scatter_wait(j, buf):
        pltpu.make_async_copy(buf, acc.at[idx_d.at[j]], sem2).wait()

    for r in range(NR):
        pltpu.sync_copy(dsti_hbm.at[s, r], idx_d)
        gather(r * RC, rows0)

        def body(jj, carry):
            j0 = 2 * jj
            g0 = r * RC + j0

            @pl.when(jj > 0)
            def _():
                scatter_wait(j0 - 1, rows1)

            gather(g0 + 1, rows1)
            gather_wait(g0, rows0)
            scatter(j0, rows0)
            scatter_wait(j0, rows0)

            @pl.when(jj < RC // 2 - 1)
            def _():
                gather(g0 + 2, rows0)

            gather_wait(g0 + 1, rows1)
            scatter(j0 + 1, rows1)
            return carry

        lax.fori_loop(0, RC // 2, body, 0)
        scatter_wait(RC - 1, rows1)

    plsc.subcore_barrier()

    @pl.when(s < 15)
    def _():
        pltpu.sync_copy(acc.at[pl.ds(row0, RPT)],
                        out_hbm.at[c, pl.ds(row0, RPT)])

    @pl.when(s == 15)
    def _():
        pltpu.sync_copy(acc.at[pl.ds(15 * RPT, RPT_LAST)],
                        out_hbm.at[c, pl.ds(15 * RPT, RPT_LAST)])


def _segsum(tbl_flat, srcs, dsti, zrows):
    return pl.kernel(
        _segsum_body,
        out_type=jax.ShapeDtypeStruct((NC, N, D), jnp.float32),
        mesh=_mesh,
        compiler_params=pltpu.CompilerParams(needs_layout_passes=False),
        scratch_types=[
            pltpu.VMEM_SHARED((N_ACC, D), jnp.float32),
            pltpu.VMEM((ETP,), jnp.int32),
            pltpu.VMEM((RC, KS), jnp.int32),
            pltpu.VMEM((KS, D), jnp.float32),
            pltpu.VMEM((KS, D), jnp.float32),
            pltpu.SemaphoreType.DMA,
            pltpu.SemaphoreType.DMA,
        ],
    )(tbl_flat, srcs, dsti, zrows)


_BN = 1000


def _input_pot_body(x_ref, sa_ref, sb_ref, wn_ref, we0_ref, we1_ref, b_ref,
                    ip_ref, tbl_ref):
    sa = jnp.sum(sa_ref[...], axis=1)[:, None]
    sb = jnp.sum(sb_ref[...], axis=1)[:, None]
    t = jnp.dot(x_ref[...], wn_ref[...], preferred_element_type=jnp.float32)
    t += sa * we0_ref[...] + sb * we1_ref[...]
    ip = jnp.maximum(t + b_ref[...], 0.0)
    ip_ref[...] = ip
    tbl_ref[0] = ip[:, :D]
    tbl_ref[1] = ip[:, D:]


def _input_potential(x, sa, sb, w_n2l, we0, we1, b_n2l):
    return pl.pallas_call(
        _input_pot_body,
        grid=(N // _BN,),
        in_specs=[
            pl.BlockSpec((_BN, D), lambda i: (i, 0)),
            pl.BlockSpec((_BN, NW), lambda i: (i, 0)),
            pl.BlockSpec((_BN, NW), lambda i: (i, 0)),
            pl.BlockSpec((D, H), lambda i: (0, 0)),
            pl.BlockSpec((1, H), lambda i: (0, 0)),
            pl.BlockSpec((1, H), lambda i: (0, 0)),
            pl.BlockSpec((1, H), lambda i: (0, 0)),
        ],
        out_specs=[
            pl.BlockSpec((_BN, H), lambda i: (i, 0)),
            pl.BlockSpec((NC, _BN, D), lambda i: (0, i, 0)),
        ],
        out_shape=[
            jax.ShapeDtypeStruct((N, H), jnp.float32),
            jax.ShapeDtypeStruct((NC, N, D), jnp.float32),
        ],
    )(x, sa, sb, w_n2l, we0, we1, b_n2l)


def _layer_body(p_ref, ip_ref, w0_ref, w1_ref, b_ref,
                tbl_ref, g_ref):
    i = pl.program_id(0)
    a0 = p_ref[0]
    a1 = p_ref[1]
    t = jnp.dot(a0, w0_ref[...], preferred_element_type=jnp.float32)
    t += jnp.dot(a1, w1_ref[...], preferred_element_type=jnp.float32)
    cur = jnp.maximum(t + b_ref[...] + ip_ref[...], 0.0)
    tbl_ref[0] = cur[:, :D]
    tbl_ref[1] = cur[:, D:]

    @pl.when(i == 0)
    def _():
        g_ref[...] = jnp.zeros_like(g_ref)

    g_ref[...] += jnp.sum(cur, axis=0, keepdims=True)


def _layer_update(parts, ip, conv_W0, conv_W1, conv_b):
    return pl.pallas_call(
        _layer_body,
        grid=(N // _BN,),
        in_specs=[
            pl.BlockSpec((NC, _BN, D), lambda i: (0, i, 0)),
            pl.BlockSpec((_BN, H), lambda i: (i, 0)),
            pl.BlockSpec((D, H), lambda i: (0, 0)),
            pl.BlockSpec((D, H), lambda i: (0, 0)),
            pl.BlockSpec((1, H), lambda i: (0, 0)),
        ],
        out_specs=[
            pl.BlockSpec((NC, _BN, D), lambda i: (0, i, 0)),
            pl.BlockSpec((1, H), lambda i: (0, 0)),
        ],
        out_shape=[
            jax.ShapeDtypeStruct((NC, N, D), jnp.float32),
            jax.ShapeDtypeStruct((1, H), jnp.float32),
        ],
    )(parts, ip, conv_W0, conv_W1, conv_b)


def _head_body(graph_ref, bud_ref, gw_ref, bw_ref, cb_ref,
               s1w_ref, s1bw_ref, s1b_ref, soutw_ref, soutb_ref,
               nw_ref, nb_ref, a1g_ref, a1ng_ref, a1b_ref,
               sv_ref, c0_ref):
    bud = bud_ref[...]
    g = jnp.dot(graph_ref[...], gw_ref[...], preferred_element_type=jnp.float32)
    g = jnp.maximum(g + bud * bw_ref[...] + cb_ref[...], 0.0)
    sh = jnp.dot(g, s1w_ref[...], preferred_element_type=jnp.float32)
    sh = jnp.maximum(sh + bud * s1bw_ref[...] + s1b_ref[...], 0.0)
    sv_ref[...] = (jnp.dot(sh, soutw_ref[...], preferred_element_type=jnp.float32)
                   + soutb_ref[...])
    ng = jnp.dot(g, nw_ref[...], preferred_element_type=jnp.float32)
    ng = jnp.maximum(ng + nb_ref[...], 0.0)
    c0 = jnp.dot(g, a1g_ref[...], preferred_element_type=jnp.float32)
    c0 += jnp.dot(ng, a1ng_ref[...], preferred_element_type=jnp.float32)
    c0_ref[...] = c0 + a1b_ref[...]


def _head(graph, budget, gW, bW, cb, s1W, s1bw, s1b, soutW, soutb,
          nodesW, nodesb, A1g, A1ng, a1b):
    return pl.pallas_call(
        _head_body,
        out_shape=[
            jax.ShapeDtypeStruct((1, 1), jnp.float32),
            jax.ShapeDtypeStruct((1, 64), jnp.float32),
        ],
    )(graph, budget, gW, bW, cb, s1W, s1bw, s1b, soutW, soutb,
      nodesW, nodesb, A1g, A1ng, a1b)


_BE = 4000


def _q_body(wt_ref, c0_ref, ww_ref, wb_ref, a1m_ref, a384_ref,
            aoutw_ref, aoutb_ref, q_ref):
    wt = wt_ref[...]
    wtemb = jnp.maximum(wt * ww_ref[...] + wb_ref[...], 0.0)
    mid = jnp.dot(wtemb, a1m_ref[...], preferred_element_type=jnp.float32)
    ah = jnp.maximum(mid + c0_ref[...] + wt * a384_ref[...], 0.0)
    q_ref[...] = (jnp.dot(ah, aoutw_ref[...], preferred_element_type=jnp.float32)
                  + aoutb_ref[...])


def _q_head(wt, c0, wW, wb, A1mid, a384, aoutW, aoutb):
    return pl.pallas_call(
        _q_body,
        grid=(E // _BE,),
        in_specs=[
            pl.BlockSpec((_BE, 1), lambda i: (i, 0)),
            pl.BlockSpec((1, 64), lambda i: (0, 0)),
            pl.BlockSpec((1, LAT), lambda i: (0, 0)),
            pl.BlockSpec((1, LAT), lambda i: (0, 0)),
            pl.BlockSpec((LAT, 64), lambda i: (0, 0)),
            pl.BlockSpec((1, 64), lambda i: (0, 0)),
            pl.BlockSpec((64, 1), lambda i: (0, 0)),
            pl.BlockSpec((1, 1), lambda i: (0, 0)),
        ],
        out_specs=pl.BlockSpec((_BE, 1), lambda i: (i, 0)),
        out_shape=jax.ShapeDtypeStruct((E, 1), jnp.float32),
    )(wt, c0, wW, wb, A1mid, a384, aoutW, aoutb)


def kernel(x, edge_index, edge_attr, budget, w_n2l, b_n2l, w_e2l, conv_W,
           conv_b, concat_W, concat_b, weight_W, weight_b, nodes_W, nodes_b,
           s1_W, s1_b, sout_W, sout_b, a1_W, a1_b, aout_W, aout_b):
    f32 = jnp.float32
    src = edge_index[0].astype(jnp.int32)
    dst = edge_index[1].astype(jnp.int32)
    srcs = jnp.concatenate([src, src + N])
    dsti = dst.reshape(NS, NR, RC, KS)

    ea0 = edge_attr[:, 0]
    ea1 = edge_attr[:, 1]
    z128 = jnp.zeros((RPT, D), f32)

    s0f, s1f = _edge_scatter(ea0, ea1, dst)
    sa = s0f.reshape(NW, N).T
    sb = s1f.reshape(NW, N).T
    ip, tbl = _input_potential(
        x, sa, sb, w_n2l, w_e2l[0].reshape(1, H), w_e2l[1].reshape(1, H),
        b_n2l.reshape(1, H))

    conv_W0 = conv_W[:D]
    conv_W1 = conv_W[D:]
    cbr = conv_b.reshape(1, H)
    graph = None
    for _ in range(3):
        pool = _segsum(tbl.reshape(NC * N, D), srcs, dsti, z128)
        tbl, graph = _layer_update(pool, ip, conv_W0, conv_W1, cbr)

    state_val, c0 = _head(
        graph, budget.reshape(1, 1),
        concat_W[:H], concat_W[H:], concat_b.reshape(1, LAT),
        s1_W[:LAT], s1_W[LAT:], s1_b.reshape(1, 64),
        sout_W, sout_b.reshape(1, 1),
        nodes_W, nodes_b.reshape(1, LAT),
        a1_W[:LAT], a1_W[2 * LAT:3 * LAT], a1_b.reshape(1, 64))

    q = _q_head(edge_attr[:, :1], c0, weight_W, weight_b.reshape(1, LAT),
                a1_W[LAT:2 * LAT], a1_W[3 * LAT:], aout_W, aout_b.reshape(1, 1))

    return jnp.concatenate([state_val, q], axis=0)


# speedup vs baseline: 1.9152x; 1.0255x over previous
import jax
import jax.numpy as jnp
from jax import lax
from jax.experimental import pallas as pl
from jax.experimental.pallas import tpu as pltpu
from jax.experimental.pallas import tpu_sc as plsc

N = 10000
E = 320000
D = 128
LAT = 128
H = 2 * LAT

NC = 2
NS = 16
NW = NC * NS
EW = E // NW
K = 80
NCHUNK = EW // K
RPT = 632
RPT_LAST = N - 15 * RPT

_mesh = plsc.VectorSubcoreMesh(core_axis_name="c", subcore_axis_name="s")


def _edge_scatter_body(ea0_hbm, ea1_hbm, dst_hbm, out0_hbm, out1_hbm,
                       acc0, acc1, dst_v, ea0_v, ea1_v):
    c = lax.axis_index("c")
    s = lax.axis_index("s")
    w = c * NS + s
    e0 = pl.multiple_of(w * EW, 8)
    pltpu.sync_copy(dst_hbm.at[pl.ds(e0, EW)], dst_v)
    pltpu.sync_copy(ea0_hbm.at[pl.ds(e0, EW)], ea0_v)
    pltpu.sync_copy(ea1_hbm.at[pl.ds(e0, EW)], ea1_v)

    z16 = jnp.zeros((16,), jnp.float32)

    def zbody(k, carry):
        acc0[pl.ds(k * 16, 16)] = z16
        acc1[pl.ds(k * 16, 16)] = z16
        return carry

    lax.fori_loop(0, N // 16, zbody, 0)

    def body(k, carry):
        o = k * 16
        idx = dst_v[pl.ds(o, 16)]
        plsc.addupdate_scatter(acc0, [idx], ea0_v[pl.ds(o, 16)])
        plsc.addupdate_scatter(acc1, [idx], ea1_v[pl.ds(o, 16)])
        return carry

    lax.fori_loop(0, EW // 16, body, 0)
    o0 = pl.multiple_of(w * N, 8)
    pltpu.sync_copy(acc0, out0_hbm.at[pl.ds(o0, N)])
    pltpu.sync_copy(acc1, out1_hbm.at[pl.ds(o0, N)])


def _edge_scatter(ea0, ea1, dst):
    return pl.kernel(
        _edge_scatter_body,
        out_type=[
            jax.ShapeDtypeStruct((NW * N,), jnp.float32),
            jax.ShapeDtypeStruct((NW * N,), jnp.float32),
        ],
        mesh=_mesh,
        compiler_params=pltpu.CompilerParams(needs_layout_passes=False),
        scratch_types=[
            pltpu.VMEM((N,), jnp.float32),
            pltpu.VMEM((N,), jnp.float32),
            pltpu.VMEM((EW,), jnp.int32),
            pltpu.VMEM((EW,), jnp.float32),
            pltpu.VMEM((EW,), jnp.float32),
        ],
    )(ea0, ea1, dst)


KS = 80
NR = 5
RC = 50
ETP = NR * RC * KS
N_ACC = N


def _segsum_body(tbl_hbm, srcs_hbm, dsti_hbm, z_hbm, out_hbm,
                 acc, idx_s, idx_d, rows0, rows1, sem, sem2):
    c = lax.axis_index("c")
    s = lax.axis_index("s")
    row0 = pl.multiple_of(s * RPT, 8)
    tile_base = pl.multiple_of((c * NS + s) * ETP, 8)
    pltpu.sync_copy(srcs_hbm.at[pl.ds(tile_base, ETP)], idx_s)

    @pl.when(s < 15)
    def _():
        pltpu.sync_copy(z_hbm.at[pl.ds(0, RPT)], acc.at[pl.ds(row0, RPT)])

    @pl.when(s == 15)
    def _():
        pltpu.sync_copy(z_hbm.at[pl.ds(0, RPT_LAST)],
                        acc.at[pl.ds(15 * RPT, RPT_LAST)])

    plsc.subcore_barrier()

    def gather(j, buf):
        jo = pl.multiple_of(j * KS, 8)
        return pltpu.async_copy(tbl_hbm.at[idx_s.at[pl.ds(jo, KS)]], buf, sem)

    def gather_wait(j, buf):
        jo = pl.multiple_of(j * KS, 8)
        pltpu.make_async_copy(tbl_hbm.at[idx_s.at[pl.ds(jo, KS)]], buf,
                              sem).wait()

    def scatter(j, buf):
        return pltpu.async_copy(buf, acc.at[idx_d.at[j]], sem2, add=True)

    def scatter_wait(j, buf):
        pltpu.make_async_copy(buf, acc.at[idx_d.at[j]], sem2).wait()

    for r in range(NR):
        pltpu.sync_copy(dsti_hbm.at[s, r], idx_d)
        gather(r * RC, rows0)

        def body(jj, carry):
            j0 = 2 * jj
            g0 = r * RC + j0

            @pl.when(jj > 0)
            def _():
                scatter_wait(j0 - 1, rows1)

            gather(g0 + 1, rows1)
            gather_wait(g0, rows0)
            scatter(j0, rows0)
            scatter_wait(j0, rows0)

            @pl.when(jj < RC // 2 - 1)
            def _():
                gather(g0 + 2, rows0)

            gather_wait(g0 + 1, rows1)
            scatter(j0 + 1, rows1)
            return carry

        lax.fori_loop(0, RC // 2, body, 0)
        scatter_wait(RC - 1, rows1)

    plsc.subcore_barrier()

    @pl.when(s < 15)
    def _():
        pltpu.sync_copy(acc.at[pl.ds(row0, RPT)],
                        out_hbm.at[c, pl.ds(row0, RPT)])

    @pl.when(s == 15)
    def _():
        pltpu.sync_copy(acc.at[pl.ds(15 * RPT, RPT_LAST)],
                        out_hbm.at[c, pl.ds(15 * RPT, RPT_LAST)])


def _segsum(tbl_flat, srcs, dsti, zrows):
    return pl.kernel(
        _segsum_body,
        out_type=jax.ShapeDtypeStruct((NC, N, D), jnp.float32),
        mesh=_mesh,
        compiler_params=pltpu.CompilerParams(needs_layout_passes=False),
        scratch_types=[
            pltpu.VMEM_SHARED((N_ACC, D), jnp.float32),
            pltpu.VMEM((ETP,), jnp.int32),
            pltpu.VMEM((RC, KS), jnp.int32),
            pltpu.VMEM((KS, D), jnp.float32),
            pltpu.VMEM((KS, D), jnp.float32),
            pltpu.SemaphoreType.DMA,
            pltpu.SemaphoreType.DMA,
        ],
    )(tbl_flat, srcs, dsti, zrows)


_BN = 1000


def _input_pot_body(x_ref, sa_ref, sb_ref, wn_ref, we0_ref, we1_ref, b_ref,
                    tbl_ref):
    sa = jnp.sum(sa_ref[...], axis=1)[:, None]
    sb = jnp.sum(sb_ref[...], axis=1)[:, None]
    t = jnp.dot(x_ref[...], wn_ref[...], preferred_element_type=jnp.float32)
    t += sa * we0_ref[...] + sb * we1_ref[...]
    ip = jnp.maximum(t + b_ref[...], 0.0)
    tbl_ref[0] = ip[:, :D]
    tbl_ref[1] = ip[:, D:]


def _input_potential(x, sa, sb, w_n2l, we0, we1, b_n2l):
    return pl.pallas_call(
        _input_pot_body,
        grid=(N // _BN,),
        in_specs=[
            pl.BlockSpec((_BN, D), lambda i: (i, 0)),
            pl.BlockSpec((_BN, NW), lambda i: (i, 0)),
            pl.BlockSpec((_BN, NW), lambda i: (i, 0)),
            pl.BlockSpec((D, H), lambda i: (0, 0)),
            pl.BlockSpec((1, H), lambda i: (0, 0)),
            pl.BlockSpec((1, H), lambda i: (0, 0)),
            pl.BlockSpec((1, H), lambda i: (0, 0)),
        ],
        out_specs=pl.BlockSpec((NC, _BN, D), lambda i: (0, i, 0)),
        out_shape=jax.ShapeDtypeStruct((NC, N, D), jnp.float32),
    )(x, sa, sb, w_n2l, we0, we1, b_n2l)


def _layer_body(p_ref, ip_ref, w0_ref, w1_ref, b_ref,
                tbl_ref, g_ref):
    i = pl.program_id(0)
    a0 = p_ref[0]
    a1 = p_ref[1]
    t = jnp.dot(a0, w0_ref[...], preferred_element_type=jnp.float32)
    t += jnp.dot(a1, w1_ref[...], preferred_element_type=jnp.float32)
    ip = jnp.concatenate([ip_ref[0], ip_ref[1]], axis=1)
    cur = jnp.maximum(t + b_ref[...] + ip, 0.0)
    tbl_ref[0] = cur[:, :D]
    tbl_ref[1] = cur[:, D:]

    @pl.when(i == 0)
    def _():
        g_ref[...] = jnp.zeros_like(g_ref)

    g_ref[...] += jnp.sum(cur, axis=0, keepdims=True)


def _layer_update(parts, ip, conv_W0, conv_W1, conv_b):
    return pl.pallas_call(
        _layer_body,
        grid=(N // _BN,),
        in_specs=[
            pl.BlockSpec((NC, _BN, D), lambda i: (0, i, 0)),
            pl.BlockSpec((NC, _BN, D), lambda i: (0, i, 0)),
            pl.BlockSpec((D, H), lambda i: (0, 0)),
            pl.BlockSpec((D, H), lambda i: (0, 0)),
            pl.BlockSpec((1, H), lambda i: (0, 0)),
        ],
        out_specs=[
            pl.BlockSpec((NC, _BN, D), lambda i: (0, i, 0)),
            pl.BlockSpec((1, H), lambda i: (0, 0)),
        ],
        out_shape=[
            jax.ShapeDtypeStruct((NC, N, D), jnp.float32),
            jax.ShapeDtypeStruct((1, H), jnp.float32),
        ],
    )(parts, ip, conv_W0, conv_W1, conv_b)


def _head_body(graph_ref, bud_ref, gw_ref, bw_ref, cb_ref,
               s1w_ref, s1bw_ref, s1b_ref, soutw_ref, soutb_ref,
               nw_ref, nb_ref, a1g_ref, a1ng_ref, a1b_ref,
               sv_ref, c0_ref):
    bud = bud_ref[...]
    g = jnp.dot(graph_ref[...], gw_ref[...], preferred_element_type=jnp.float32)
    g = jnp.maximum(g + bud * bw_ref[...] + cb_ref[...], 0.0)
    sh = jnp.dot(g, s1w_ref[...], preferred_element_type=jnp.float32)
    sh = jnp.maximum(sh + bud * s1bw_ref[...] + s1b_ref[...], 0.0)
    sv_ref[...] = (jnp.dot(sh, soutw_ref[...], preferred_element_type=jnp.float32)
                   + soutb_ref[...])
    ng = jnp.dot(g, nw_ref[...], preferred_element_type=jnp.float32)
    ng = jnp.maximum(ng + nb_ref[...], 0.0)
    c0 = jnp.dot(g, a1g_ref[...], preferred_element_type=jnp.float32)
    c0 += jnp.dot(ng, a1ng_ref[...], preferred_element_type=jnp.float32)
    c0_ref[...] = c0 + a1b_ref[...]


def _head(graph, budget, gW, bW, cb, s1W, s1bw, s1b, soutW, soutb,
          nodesW, nodesb, A1g, A1ng, a1b):
    return pl.pallas_call(
        _head_body,
        out_shape=[
            jax.ShapeDtypeStruct((1, 1), jnp.float32),
            jax.ShapeDtypeStruct((1, 64), jnp.float32),
        ],
    )(graph, budget, gW, bW, cb, s1W, s1bw, s1b, soutW, soutb,
      nodesW, nodesb, A1g, A1ng, a1b)


_BE = 8000


def _q_body(wt_ref, c0_ref, ww_ref, wb_ref, a1m_ref, a384_ref,
            aoutw_ref, aoutb_ref, q_ref):
    wt = wt_ref[...]
    wtemb = jnp.maximum(wt * ww_ref[...] + wb_ref[...], 0.0)
    mid = jnp.dot(wtemb, a1m_ref[...], preferred_element_type=jnp.float32)
    ah = jnp.maximum(mid + c0_ref[...] + wt * a384_ref[...], 0.0)
    q_ref[...] = (jnp.dot(ah, aoutw_ref[...], preferred_element_type=jnp.float32)
                  + aoutb_ref[...])


def _q_head(wt, c0, wW, wb, A1mid, a384, aoutW, aoutb):
    return pl.pallas_call(
        _q_body,
        grid=(E // _BE,),
        in_specs=[
            pl.BlockSpec((_BE, 1), lambda i: (i, 0)),
            pl.BlockSpec((1, 64), lambda i: (0, 0)),
            pl.BlockSpec((1, LAT), lambda i: (0, 0)),
            pl.BlockSpec((1, LAT), lambda i: (0, 0)),
            pl.BlockSpec((LAT, 64), lambda i: (0, 0)),
            pl.BlockSpec((1, 64), lambda i: (0, 0)),
            pl.BlockSpec((64, 1), lambda i: (0, 0)),
            pl.BlockSpec((1, 1), lambda i: (0, 0)),
        ],
        out_specs=pl.BlockSpec((_BE, 1), lambda i: (i, 0)),
        out_shape=jax.ShapeDtypeStruct((E, 1), jnp.float32),
    )(wt, c0, wW, wb, A1mid, a384, aoutW, aoutb)


def kernel(x, edge_index, edge_attr, budget, w_n2l, b_n2l, w_e2l, conv_W,
           conv_b, concat_W, concat_b, weight_W, weight_b, nodes_W, nodes_b,
           s1_W, s1_b, sout_W, sout_b, a1_W, a1_b, aout_W, aout_b):
    f32 = jnp.float32
    src = edge_index[0].astype(jnp.int32)
    dst = edge_index[1].astype(jnp.int32)
    srcs = jnp.concatenate([src, src + N])
    dsti = dst.reshape(NS, NR, RC, KS)

    ea0 = edge_attr[:, 0]
    ea1 = edge_attr[:, 1]
    z128 = jnp.zeros((RPT, D), f32)

    s0f, s1f = _edge_scatter(ea0, ea1, dst)
    sa = s0f.reshape(NW, N).T
    sb = s1f.reshape(NW, N).T
    tbl0 = _input_potential(
        x, sa, sb, w_n2l, w_e2l[0].reshape(1, H), w_e2l[1].reshape(1, H),
        b_n2l.reshape(1, H))
    tbl = tbl0

    conv_W0 = conv_W[:D]
    conv_W1 = conv_W[D:]
    cbr = conv_b.reshape(1, H)
    graph = None
    for _ in range(3):
        pool = _segsum(tbl.reshape(NC * N, D), srcs, dsti, z128)
        tbl, graph = _layer_update(pool, tbl0, conv_W0, conv_W1, cbr)

    state_val, c0 = _head(
        graph, budget.reshape(1, 1),
        concat_W[:H], concat_W[H:], concat_b.reshape(1, LAT),
        s1_W[:LAT], s1_W[LAT:], s1_b.reshape(1, 64),
        sout_W, sout_b.reshape(1, 1),
        nodes_W, nodes_b.reshape(1, LAT),
        a1_W[:LAT], a1_W[2 * LAT:3 * LAT], a1_b.reshape(1, 64))

    q = _q_head(edge_attr[:, :1], c0, weight_W, weight_b.reshape(1, LAT),
                a1_W[LAT:2 * LAT], a1_W[3 * LAT:], aout_W, aout_b.reshape(1, 1))

    return jnp.concatenate([state_val, q], axis=0)
